# Initial kernel scaffold; baseline (speedup 1.0000x reference)
import jax
import jax.numpy as jnp
from jax.experimental import pallas as pl


def kernel(x, edge_index, W1, a_src1, a_dst1, W2, a_src2, a_dst2):
    raise NotImplementedError("write your pallas kernel here")


# speedup vs baseline: 27.7569x; 27.7569x over previous
import functools

import jax
import jax.numpy as jnp
from jax import lax
from jax.experimental import pallas as pl
from jax.experimental.pallas import tpu as pltpu
from jax.experimental.pallas import tpu_sc as plsc

N = 10000
E = 320000
D_IN = 128
H = 8
F1 = 64
F2 = 128
NTILES = 32
EPT = E // NTILES
B = 80
NB = EPT // B
ROWS_PT = 624
ROWS_REM = N - 16 * ROWS_PT

f32 = jnp.float32
i32 = jnp.int32


def _lrelu_exp(s):
    return jnp.exp(jnp.where(s >= 0, s, 0.2 * s))


def _tc1_body(x_ref, w_ref, as_ref, ad_ref, em_ref,
              h_ref, asrc_ref, adst_ref, den0_ref, acc0_ref):
    h = jnp.dot(x_ref[...], w_ref[...], preferred_element_type=f32)
    asrc = jnp.dot(h, as_ref[...], preferred_element_type=f32)
    adst = jnp.dot(h, ad_ref[...], preferred_element_type=f32)
    ws = _lrelu_exp(asrc + adst)
    h_ref[...] = h
    asrc_ref[...] = asrc
    adst_ref[...] = adst
    den0_ref[...] = 0.5 * ws
    acc0_ref[...] = 0.5 * (jnp.dot(ws, em_ref[...], preferred_element_type=f32) * h)


def _tc2_body(denp_ref, accp_ref, em1_ref, w2_ref, as2_ref, ad2_ref, em2_ref,
              h2_ref, asrc_ref, adst_ref, den0_ref, acc0_ref):
    denp = denp_ref[...]
    accp = accp_ref[...]
    den = denp[0] + denp[1] + 1e-16
    acc = accp[0] + accp[1]
    out1 = acc / jnp.dot(den, em1_ref[...], preferred_element_type=f32)
    hh = jnp.where(out1 > 0, out1, jnp.exp(out1) - 1.0)
    h2 = jnp.dot(hh, w2_ref[...], preferred_element_type=f32)
    asrc = jnp.dot(h2, as2_ref[...], preferred_element_type=f32)
    adst = jnp.dot(h2, ad2_ref[...], preferred_element_type=f32)
    ws = _lrelu_exp(asrc + adst)
    h2_ref[...] = h2
    asrc_ref[...] = asrc
    adst_ref[...] = adst
    den0_ref[...] = 0.5 * ws
    acc0_ref[...] = 0.5 * (jnp.dot(ws, em2_ref[...], preferred_element_type=f32) * h2)


def _tc3_body(denp_ref, accp_ref, em2_ref, o_ref):
    denp = denp_ref[...]
    accp = accp_ref[...]
    den = denp[0] + denp[1] + 1e-16
    acc = accp[0] + accp[1]
    out = acc / jnp.dot(den, em2_ref[...], preferred_element_type=f32)
    m = jnp.max(out, axis=-1, keepdims=True)
    z = out - m
    lse = jnp.log(jnp.sum(jnp.exp(z), axis=-1, keepdims=True))
    o_ref[...] = z - lse


_R = 2000


def _rows(shape):
    return pl.BlockSpec((_R, shape), lambda i: (i, 0))


def _full(*shape):
    return pl.BlockSpec(shape, lambda i: (0,) * len(shape))


def _prow(k):
    return pl.BlockSpec((2, _R, k), lambda i: (0, i, 0))


def _tc1(x, w1, as1, ad1, em1):
    return pl.pallas_call(
        _tc1_body,
        grid=(N // _R,),
        in_specs=[_rows(D_IN), _full(D_IN, F1), _full(F1, H), _full(F1, H),
                  _full(H, F1)],
        out_specs=[_rows(F1), _rows(H), _rows(H), _rows(H), _rows(F1)],
        out_shape=[
            jax.ShapeDtypeStruct((N, F1), f32),
            jax.ShapeDtypeStruct((N, H), f32),
            jax.ShapeDtypeStruct((N, H), f32),
            jax.ShapeDtypeStruct((N, H), f32),
            jax.ShapeDtypeStruct((N, F1), f32),
        ],
    )(x, w1, as1, ad1, em1)


def _tc2(denp, accp, em1, w2, as2, ad2, em2):
    return pl.pallas_call(
        _tc2_body,
        grid=(N // _R,),
        in_specs=[_prow(H), _prow(F1), _full(H, F1), _full(F1, F2),
                  _full(F2, H), _full(F2, H), _full(H, F2)],
        out_specs=[_rows(F2), _rows(H), _rows(H), _rows(H), _rows(F2)],
        out_shape=[
            jax.ShapeDtypeStruct((N, F2), f32),
            jax.ShapeDtypeStruct((N, H), f32),
            jax.ShapeDtypeStruct((N, H), f32),
            jax.ShapeDtypeStruct((N, H), f32),
            jax.ShapeDtypeStruct((N, F2), f32),
        ],
    )(denp, accp, em1, w2, as2, ad2, em2)


def _tc3(denp, accp, em2):
    return pl.pallas_call(
        _tc3_body,
        grid=(N // _R,),
        in_specs=[_prow(H), _prow(F2), _full(H, F2)],
        out_specs=[_rows(F2)],
        out_shape=[jax.ShapeDtypeStruct((N, F2), f32)],
    )(denp, accp, em2)


def _make_sc_layer(F):
    lgF = F.bit_length() - 1
    lgG = (F // 8).bit_length() - 1

    mesh = plsc.VectorSubcoreMesh(core_axis_name="c", subcore_axis_name="s")

    @functools.partial(
        pl.kernel,
        mesh=mesh,
        compiler_params=pltpu.CompilerParams(
            needs_layout_passes=False,
            use_tc_tiling_on_sc=False,
        ),
        out_type=[
            jax.ShapeDtypeStruct((2, N, H), f32),
            jax.ShapeDtypeStruct((2, N, F), f32),
        ],
        scratch_types=[
            pltpu.VMEM_SHARED((N, H), f32),
            pltpu.VMEM_SHARED((N, F), f32),
            pltpu.VMEM((NB, B), i32),
            pltpu.VMEM((NB, B), i32),
            pltpu.VMEM((B, H), f32),
            pltpu.VMEM((B, H), f32),
            pltpu.VMEM((B, H), f32),
            pltpu.VMEM((B, F), f32),
        ],
    )
    def sc_layer(src_hbm, dst_hbm, asrc_hbm, adst_hbm, h_hbm, den0_hbm,
                 acc0_hbm, den_out, acc_out,
                 sh_den, sh_acc, src_buf, dst_buf, a_s, a_d, w2d, hrows):
        cid = lax.axis_index("c")
        sid = lax.axis_index("s")
        wid = cid * 16 + sid
        e0 = wid * EPT
        r0 = sid * ROWS_PT

        pltpu.sync_copy(den0_hbm.at[pl.ds(r0, ROWS_PT)],
                        sh_den.at[pl.ds(r0, ROWS_PT)])
        pltpu.sync_copy(acc0_hbm.at[pl.ds(r0, ROWS_PT)],
                        sh_acc.at[pl.ds(r0, ROWS_PT)])

        @pl.when(sid == 15)
        def _stage_rem():
            pltpu.sync_copy(den0_hbm.at[pl.ds(16 * ROWS_PT, ROWS_REM)],
                            sh_den.at[pl.ds(16 * ROWS_PT, ROWS_REM)])
            pltpu.sync_copy(acc0_hbm.at[pl.ds(16 * ROWS_PT, ROWS_REM)],
                            sh_acc.at[pl.ds(16 * ROWS_PT, ROWS_REM)])

        plsc.subcore_barrier()

        iota = lax.iota(i32, 16)
        koff = iota >> 3
        kcol = iota & 7

        def block(b, carry):
            eb = pl.multiple_of(e0 + b * B, 16)
            pltpu.sync_copy(src_hbm.at[pl.ds(eb, B)], src_buf.at[b])
            pltpu.sync_copy(dst_hbm.at[pl.ds(eb, B)], dst_buf.at[b])
            pltpu.sync_copy(asrc_hbm.at[src_buf.at[b]], a_s)
            pltpu.sync_copy(adst_hbm.at[dst_buf.at[b]], a_d)
            pltpu.sync_copy(h_hbm.at[src_buf.at[b]], hrows)

            def p1(j, c1):
                rows = 2 * j + koff
                av = plsc.load_gather(a_s, [rows, kcol])
                dv = plsc.load_gather(a_d, [rows, kcol])
                plsc.store_scatter(w2d, [rows, kcol], _lrelu_exp(av + dv))
                return c1

            lax.fori_loop(0, B * H // 16, p1, 0)
            pltpu.sync_copy(w2d, sh_den.at[dst_buf.at[b]], add=True)

            def p2(j, c2):
                t = 16 * j + iota
                hv = plsc.load_gather(hrows, [t >> lgF, t & (F - 1)])
                wi = t >> lgG
                wv = plsc.load_gather(w2d, [wi >> 3, wi & 7])
                plsc.store_scatter(hrows, [t >> lgF, t & (F - 1)], hv * wv)
                return c2

            lax.fori_loop(0, B * F // 16, p2, 0)
            pltpu.sync_copy(hrows, sh_acc.at[dst_buf.at[b]], add=True)
            return carry

        lax.fori_loop(0, NB, block, 0)
        plsc.subcore_barrier()

        pltpu.sync_copy(sh_den.at[pl.ds(r0, ROWS_PT)],
                        den_out.at[cid, pl.ds(r0, ROWS_PT)])
        pltpu.sync_copy(sh_acc.at[pl.ds(r0, ROWS_PT)],
                        acc_out.at[cid, pl.ds(r0, ROWS_PT)])

        @pl.when(sid == 15)
        def _out_rem():
            pltpu.sync_copy(sh_den.at[pl.ds(16 * ROWS_PT, ROWS_REM)],
                            den_out.at[cid, pl.ds(16 * ROWS_PT, ROWS_REM)])
            pltpu.sync_copy(sh_acc.at[pl.ds(16 * ROWS_PT, ROWS_REM)],
                            acc_out.at[cid, pl.ds(16 * ROWS_PT, ROWS_REM)])

    return sc_layer


_sc1 = _make_sc_layer(F1)
_sc2 = _make_sc_layer(F2)


def _pack_a(a_flat, heads, ch):
    d = heads * ch
    rows = jnp.arange(d)[:, None] // ch
    mask = (rows == jnp.arange(8)[None, :]).astype(f32)
    return a_flat[:, None] * mask


def _expand_mat(heads, ch):
    d = heads * ch
    cols = jnp.arange(d)[None, :] // ch
    return (jnp.arange(8)[:, None] == cols).astype(f32)


def kernel(x, edge_index, W1, a_src1, a_dst1, W2, a_src2, a_dst2):
    src = edge_index[0]
    dst = edge_index[1]

    as1 = _pack_a(a_src1.reshape(-1), H, F1 // H)
    ad1 = _pack_a(a_dst1.reshape(-1), H, F1 // H)
    em1 = _expand_mat(H, F1 // H)
    as2 = jnp.tile(a_src2.reshape(F2, 1), (1, 8))
    ad2 = jnp.tile(a_dst2.reshape(F2, 1), (1, 8))
    em2 = _expand_mat(1, F2)[:8]

    h1, asrc1, adst1, den0h, acc0h = _tc1(x, W1, as1, ad1, em1)
    denp1, accp1 = _sc1(src, dst, asrc1, adst1, h1, den0h, acc0h)
    h2, asrc2, adst2, den02h, acc02h = _tc2(denp1, accp1, em1, W2, as2, ad2, em2)
    denp2, accp2 = _sc2(src, dst, asrc2, adst2, h2, den02h, acc02h)
    return _tc3(denp2, accp2, em2)[0]


# speedup vs baseline: 46.6569x; 1.6809x over previous
import functools

import jax
import jax.numpy as jnp
from jax import lax
from jax.experimental import pallas as pl
from jax.experimental.pallas import tpu as pltpu
from jax.experimental.pallas import tpu_sc as plsc

N = 10000
E = 320000
D_IN = 128
H = 8
F1 = 64
F2 = 128
NTILES = 32
EPT = E // NTILES
B = 100
NSB = 10
NBS = 10
NB = NSB * NBS
ROWS_PT = 624
ROWS_REM = N - 16 * ROWS_PT

f32 = jnp.float32
i32 = jnp.int32


def _lrelu_exp(s):
    return jnp.exp(jnp.where(s >= 0, s, 0.2 * s))


def _tc1_body(x_ref, w_ref, as_ref, ad_ref, em_ref,
              h_ref, asrc_ref, adst_ref, den0_ref, acc0_ref):
    h = jnp.dot(x_ref[...], w_ref[...], preferred_element_type=f32)
    asrc = jnp.dot(h, as_ref[...], preferred_element_type=f32)
    adst = jnp.dot(h, ad_ref[...], preferred_element_type=f32)
    ws = _lrelu_exp(asrc + adst)
    h_ref[...] = h
    asrc_ref[...] = asrc
    adst_ref[...] = adst
    den0_ref[...] = 0.5 * ws
    acc0_ref[...] = 0.5 * (jnp.dot(ws, em_ref[...], preferred_element_type=f32) * h)


def _tc2_body(denp_ref, accp_ref, em1_ref, w2_ref, as2_ref, ad2_ref, em2_ref,
              h2_ref, asrc_ref, adst_ref, den0_ref, acc0_ref):
    denp = denp_ref[...]
    accp = accp_ref[...]
    den = denp[0] + denp[1] + 1e-16
    acc = accp[0] + accp[1]
    out1 = acc / jnp.dot(den, em1_ref[...], preferred_element_type=f32)
    hh = jnp.where(out1 > 0, out1, jnp.exp(out1) - 1.0)
    h2 = jnp.dot(hh, w2_ref[...], preferred_element_type=f32)
    asrc = jnp.dot(h2, as2_ref[...], preferred_element_type=f32)
    adst = jnp.dot(h2, ad2_ref[...], preferred_element_type=f32)
    ws = _lrelu_exp(asrc + adst)
    h2_ref[...] = h2
    asrc_ref[...] = asrc
    adst_ref[...] = adst
    den0_ref[...] = 0.5 * ws
    acc0_ref[...] = 0.5 * (jnp.dot(ws, em2_ref[...], preferred_element_type=f32) * h2)


def _tc3_body(denp_ref, accp_ref, em2_ref, o_ref):
    denp = denp_ref[...]
    accp = accp_ref[...]
    den = denp[0] + denp[1] + 1e-16
    acc = accp[0] + accp[1]
    out = acc / jnp.dot(den, em2_ref[...], preferred_element_type=f32)
    m = jnp.max(out, axis=-1, keepdims=True)
    z = out - m
    lse = jnp.log(jnp.sum(jnp.exp(z), axis=-1, keepdims=True))
    o_ref[...] = z - lse


_R = 2000


def _rows(shape):
    return pl.BlockSpec((_R, shape), lambda i: (i, 0))


def _full(*shape):
    return pl.BlockSpec(shape, lambda i: (0,) * len(shape))


def _prow(k):
    return pl.BlockSpec((2, _R, k), lambda i: (0, i, 0))


def _tc1(x, w1, as1, ad1, em1):
    return pl.pallas_call(
        _tc1_body,
        grid=(N // _R,),
        in_specs=[_rows(D_IN), _full(D_IN, F1), _full(F1, H), _full(F1, H),
                  _full(H, F1)],
        out_specs=[_rows(F1), _rows(H), _rows(H), _rows(H), _rows(F1)],
        out_shape=[
            jax.ShapeDtypeStruct((N, F1), f32),
            jax.ShapeDtypeStruct((N, H), f32),
            jax.ShapeDtypeStruct((N, H), f32),
            jax.ShapeDtypeStruct((N, H), f32),
            jax.ShapeDtypeStruct((N, F1), f32),
        ],
    )(x, w1, as1, ad1, em1)


def _tc2(denp, accp, em1, w2, as2, ad2, em2):
    return pl.pallas_call(
        _tc2_body,
        grid=(N // _R,),
        in_specs=[_prow(H), _prow(F1), _full(H, F1), _full(F1, F2),
                  _full(F2, H), _full(F2, H), _full(H, F2)],
        out_specs=[_rows(F2), _rows(H), _rows(H), _rows(H), _rows(F2)],
        out_shape=[
            jax.ShapeDtypeStruct((N, F2), f32),
            jax.ShapeDtypeStruct((N, H), f32),
            jax.ShapeDtypeStruct((N, H), f32),
            jax.ShapeDtypeStruct((N, H), f32),
            jax.ShapeDtypeStruct((N, F2), f32),
        ],
    )(denp, accp, em1, w2, as2, ad2, em2)


def _tc3(denp, accp, em2):
    return pl.pallas_call(
        _tc3_body,
        grid=(N // _R,),
        in_specs=[_prow(H), _prow(F2), _full(H, F2)],
        out_specs=[_rows(F2)],
        out_shape=[jax.ShapeDtypeStruct((N, F2), f32)],
    )(denp, accp, em2)


def _make_sc_layer(F):
    lgF = F.bit_length() - 1
    lgG = (F // 8).bit_length() - 1

    mesh = plsc.VectorSubcoreMesh(core_axis_name="c", subcore_axis_name="s")

    @functools.partial(
        pl.kernel,
        mesh=mesh,
        compiler_params=pltpu.CompilerParams(
            needs_layout_passes=False,
            use_tc_tiling_on_sc=False,
        ),
        out_type=[
            jax.ShapeDtypeStruct((2, N, H), f32),
            jax.ShapeDtypeStruct((2, N, F), f32),
        ],
        scratch_types=[
            pltpu.VMEM_SHARED((N, H), f32),
            pltpu.VMEM_SHARED((N, F), f32),
            pltpu.VMEM((NBS, B), i32),
            pltpu.VMEM((NBS, B), i32),
            [pltpu.VMEM((B, H), f32)] * 2,
            [pltpu.VMEM((B, H), f32)] * 2,
            [pltpu.VMEM((B, H), f32)] * 2,
            [pltpu.VMEM((B, F), f32)] * 2,
            [pltpu.SemaphoreType.DMA] * 2,
            [pltpu.SemaphoreType.DMA] * 2,
        ],
    )
    def sc_layer(src_hbm, dst_hbm, asrc_hbm, adst_hbm, h_hbm, den0_hbm,
                 acc0_hbm, den_out, acc_out,
                 sh_den, sh_acc, src_buf, dst_buf, a_s, a_d, w2d, hrows,
                 gsem, ssem):
        cid = lax.axis_index("c")
        sid = lax.axis_index("s")
        wid = cid * 16 + sid
        r0 = sid * ROWS_PT

        pltpu.sync_copy(den0_hbm.at[pl.ds(r0, ROWS_PT)],
                        sh_den.at[pl.ds(r0, ROWS_PT)])
        pltpu.sync_copy(acc0_hbm.at[pl.ds(r0, ROWS_PT)],
                        sh_acc.at[pl.ds(r0, ROWS_PT)])

        @pl.when(sid == 15)
        def _stage_rem():
            pltpu.sync_copy(den0_hbm.at[pl.ds(16 * ROWS_PT, ROWS_REM)],
                            sh_den.at[pl.ds(16 * ROWS_PT, ROWS_REM)])
            pltpu.sync_copy(acc0_hbm.at[pl.ds(16 * ROWS_PT, ROWS_REM)],
                            sh_acc.at[pl.ds(16 * ROWS_PT, ROWS_REM)])

        plsc.subcore_barrier()

        iota = lax.iota(i32, 16)
        koff = iota >> 3
        kcol = iota & 7

        def gathers(k, p):
            return [
                pltpu.async_copy(asrc_hbm.at[src_buf.at[k]], a_s[p], gsem[p]),
                pltpu.async_copy(adst_hbm.at[dst_buf.at[k]], a_d[p], gsem[p]),
                pltpu.async_copy(h_hbm.at[src_buf.at[k]], hrows[p], gsem[p]),
            ]

        def compute(p):
            def p1(j, c1):
                rows = 2 * j + koff
                av = plsc.load_gather(a_s[p], [rows, kcol])
                dv = plsc.load_gather(a_d[p], [rows, kcol])
                plsc.store_scatter(w2d[p], [rows, kcol], _lrelu_exp(av + dv))
                return c1

            lax.fori_loop(0, B * H // 16, p1, 0, unroll=2)

            def p2(j, c2):
                t = 16 * j + iota
                hv = plsc.load_gather(hrows[p], [t >> lgF, t & (F - 1)])
                wi = t >> lgG
                wv = plsc.load_gather(w2d[p], [wi >> 3, wi & 7])
                plsc.store_scatter(hrows[p], [t >> lgF, t & (F - 1)], hv * wv)
                return c2

            lax.fori_loop(0, B * F // 16, p2, 0, unroll=4)

        def scatters(k, p):
            return [
                pltpu.async_copy(w2d[p], sh_den.at[dst_buf.at[k]], ssem[p],
                                 add=True),
                pltpu.async_copy(hrows[p], sh_acc.at[dst_buf.at[k]], ssem[p],
                                 add=True),
            ]

        def superblock(d, carry):
            pltpu.sync_copy(src_hbm.at[wid, d], src_buf)
            pltpu.sync_copy(dst_hbm.at[wid, d], dst_buf)
            g = {0: gathers(0, 0)}
            s = {}
            for k in range(NBS):
                p = k % 2
                for c in g.pop(k):
                    c.wait()
                if k >= 1:
                    for c in s.pop(k - 1):
                        c.wait()
                if k + 1 < NBS:
                    g[k + 1] = gathers(k + 1, (k + 1) % 2)
                compute(p)
                s[k] = scatters(k, p)
            for c in s.pop(NBS - 1):
                c.wait()
            return carry

        lax.fori_loop(0, NSB, superblock, 0)
        plsc.subcore_barrier()

        pltpu.sync_copy(sh_den.at[pl.ds(r0, ROWS_PT)],
                        den_out.at[cid, pl.ds(r0, ROWS_PT)])
        pltpu.sync_copy(sh_acc.at[pl.ds(r0, ROWS_PT)],
                        acc_out.at[cid, pl.ds(r0, ROWS_PT)])

        @pl.when(sid == 15)
        def _out_rem():
            pltpu.sync_copy(sh_den.at[pl.ds(16 * ROWS_PT, ROWS_REM)],
                            den_out.at[cid, pl.ds(16 * ROWS_PT, ROWS_REM)])
            pltpu.sync_copy(sh_acc.at[pl.ds(16 * ROWS_PT, ROWS_REM)],
                            acc_out.at[cid, pl.ds(16 * ROWS_PT, ROWS_REM)])

    return sc_layer


_sc1 = _make_sc_layer(F1)
_sc2 = _make_sc_layer(F2)


def _pack_a(a_flat, heads, ch):
    d = heads * ch
    rows = jnp.arange(d)[:, None] // ch
    mask = (rows == jnp.arange(8)[None, :]).astype(f32)
    return a_flat[:, None] * mask


def _expand_mat(heads, ch):
    d = heads * ch
    cols = jnp.arange(d)[None, :] // ch
    return (jnp.arange(8)[:, None] == cols).astype(f32)


def kernel(x, edge_index, W1, a_src1, a_dst1, W2, a_src2, a_dst2):
    src = edge_index[0].reshape(NTILES, NSB, NBS, B)
    dst = edge_index[1].reshape(NTILES, NSB, NBS, B)

    as1 = _pack_a(a_src1.reshape(-1), H, F1 // H)
    ad1 = _pack_a(a_dst1.reshape(-1), H, F1 // H)
    em1 = _expand_mat(H, F1 // H)
    as2 = jnp.tile(a_src2.reshape(F2, 1), (1, 8))
    ad2 = jnp.tile(a_dst2.reshape(F2, 1), (1, 8))
    em2 = _expand_mat(1, F2)[:8]

    h1, asrc1, adst1, den0h, acc0h = _tc1(x, W1, as1, ad1, em1)
    denp1, accp1 = _sc1(src, dst, asrc1, adst1, h1, den0h, acc0h)
    h2, asrc2, adst2, den02h, acc02h = _tc2(denp1, accp1, em1, W2, as2, ad2, em2)
    denp2, accp2 = _sc2(src, dst, asrc2, adst2, h2, den02h, acc02h)
    return _tc3(denp2, accp2, em2)[0]


# speedup vs baseline: 113.1303x; 2.4247x over previous
import functools

import jax
import jax.numpy as jnp
from jax import lax
from jax.experimental import pallas as pl
from jax.experimental.pallas import tpu as pltpu
from jax.experimental.pallas import tpu_sc as plsc

N = 10000
E = 320000
D_IN = 128
H = 8
F1 = 64
F2 = 128
NTILES = 32
EPT = E // NTILES
B = 100
NSB = 10
NBS = 10
NB = NSB * NBS
ROWS_PT = 624
ROWS_REM = N - 16 * ROWS_PT

f32 = jnp.float32
i32 = jnp.int32


def _lrelu_exp(s):
    return jnp.exp(jnp.where(s >= 0, s, 0.2 * s))


def _tc1_body(x_ref, w_ref, as_ref, ad_ref, em_ref,
              h_ref, asrc_ref, adst_ref, den0_ref, acc0_ref):
    h = jnp.dot(x_ref[...], w_ref[...], preferred_element_type=f32)
    asrc = jnp.dot(h, as_ref[...], preferred_element_type=f32)
    adst = jnp.dot(h, ad_ref[...], preferred_element_type=f32)
    ws = _lrelu_exp(asrc + adst)
    h_ref[...] = h
    asrc_ref[...] = asrc
    adst_ref[...] = adst
    den0_ref[...] = 0.5 * ws
    acc0_ref[...] = 0.5 * (jnp.dot(ws, em_ref[...], preferred_element_type=f32) * h)


def _tc2_body(denp_ref, accp_ref, em1_ref, w2_ref, as2_ref, ad2_ref, em2_ref,
              h2_ref, asrc_ref, adst_ref, den0_ref, acc0_ref):
    denp = denp_ref[...]
    accp = accp_ref[...]
    den = denp[0] + denp[1] + 1e-16
    acc = accp[0] + accp[1]
    out1 = acc / jnp.dot(den, em1_ref[...], preferred_element_type=f32)
    hh = jnp.where(out1 > 0, out1, jnp.exp(out1) - 1.0)
    h2 = jnp.dot(hh, w2_ref[...], preferred_element_type=f32)
    asrc = jnp.dot(h2, as2_ref[...], preferred_element_type=f32)
    adst = jnp.dot(h2, ad2_ref[...], preferred_element_type=f32)
    ws = _lrelu_exp(asrc + adst)
    h2_ref[...] = h2
    asrc_ref[...] = asrc
    adst_ref[...] = adst
    den0_ref[...] = 0.5 * ws
    acc0_ref[...] = 0.5 * (jnp.dot(ws, em2_ref[...], preferred_element_type=f32) * h2)


def _tc3_body(denp_ref, accp_ref, em2_ref, o_ref):
    denp = denp_ref[...]
    accp = accp_ref[...]
    den = denp[0] + denp[1] + 1e-16
    acc = accp[0] + accp[1]
    out = acc / jnp.dot(den, em2_ref[...], preferred_element_type=f32)
    m = jnp.max(out, axis=-1, keepdims=True)
    z = out - m
    lse = jnp.log(jnp.sum(jnp.exp(z), axis=-1, keepdims=True))
    o_ref[...] = z - lse


_R = 2000


def _rows(shape):
    return pl.BlockSpec((_R, shape), lambda i: (i, 0))


def _full(*shape):
    return pl.BlockSpec(shape, lambda i: (0,) * len(shape))


def _prow(k):
    return pl.BlockSpec((2, _R, k), lambda i: (0, i, 0))


def _tc1(x, w1, as1, ad1, em1):
    return pl.pallas_call(
        _tc1_body,
        grid=(N // _R,),
        in_specs=[_rows(D_IN), _full(D_IN, F1), _full(F1, H), _full(F1, H),
                  _full(H, F1)],
        out_specs=[_rows(F1), _rows(H), _rows(H), _rows(H), _rows(F1)],
        out_shape=[
            jax.ShapeDtypeStruct((N, F1), f32),
            jax.ShapeDtypeStruct((N, H), f32),
            jax.ShapeDtypeStruct((N, H), f32),
            jax.ShapeDtypeStruct((N, H), f32),
            jax.ShapeDtypeStruct((N, F1), f32),
        ],
    )(x, w1, as1, ad1, em1)


def _tc2(denp, accp, em1, w2, as2, ad2, em2):
    return pl.pallas_call(
        _tc2_body,
        grid=(N // _R,),
        in_specs=[_prow(H), _prow(F1), _full(H, F1), _full(F1, F2),
                  _full(F2, H), _full(F2, H), _full(H, F2)],
        out_specs=[_rows(F2), _rows(H), _rows(H), _rows(H), _rows(F2)],
        out_shape=[
            jax.ShapeDtypeStruct((N, F2), f32),
            jax.ShapeDtypeStruct((N, H), f32),
            jax.ShapeDtypeStruct((N, H), f32),
            jax.ShapeDtypeStruct((N, H), f32),
            jax.ShapeDtypeStruct((N, F2), f32),
        ],
    )(denp, accp, em1, w2, as2, ad2, em2)


def _tc3(denp, accp, em2):
    return pl.pallas_call(
        _tc3_body,
        grid=(N // _R,),
        in_specs=[_prow(H), _prow(F2), _full(H, F2)],
        out_specs=[_rows(F2)],
        out_shape=[jax.ShapeDtypeStruct((N, F2), f32)],
    )(denp, accp, em2)


def _make_sc_layer(F):
    lgF = F.bit_length() - 1
    lgG = (F // 8).bit_length() - 1

    mesh = plsc.VectorSubcoreMesh(core_axis_name="c", subcore_axis_name="s")

    @functools.partial(
        pl.kernel,
        mesh=mesh,
        compiler_params=pltpu.CompilerParams(
            needs_layout_passes=False,
            use_tc_tiling_on_sc=False,
        ),
        out_type=[
            jax.ShapeDtypeStruct((2, N, H), f32),
            jax.ShapeDtypeStruct((2, N, F), f32),
        ],
        scratch_types=[
            pltpu.VMEM_SHARED((N, H), f32),
            pltpu.VMEM_SHARED((N, F), f32),
            pltpu.VMEM((NBS, B), i32),
            pltpu.VMEM((NBS, B), i32),
            [pltpu.VMEM((B, H), f32)] * 2,
            [pltpu.VMEM((B, H), f32)] * 2,
            [pltpu.VMEM((B, H), f32)] * 2,
            [pltpu.VMEM((B, F), f32)] * 2,
            [pltpu.SemaphoreType.DMA] * 2,
            [pltpu.SemaphoreType.DMA] * 2,
        ],
    )
    def sc_layer(src_hbm, dst_hbm, asrc_hbm, adst_hbm, h_hbm, den0_hbm,
                 acc0_hbm, den_out, acc_out,
                 sh_den, sh_acc, src_buf, dst_buf, a_s, a_d, w2d, hrows,
                 gsem, ssem):
        cid = lax.axis_index("c")
        sid = lax.axis_index("s")
        wid = cid * 16 + sid
        r0 = sid * ROWS_PT

        pltpu.sync_copy(den0_hbm.at[pl.ds(r0, ROWS_PT)],
                        sh_den.at[pl.ds(r0, ROWS_PT)])
        pltpu.sync_copy(acc0_hbm.at[pl.ds(r0, ROWS_PT)],
                        sh_acc.at[pl.ds(r0, ROWS_PT)])

        @pl.when(sid == 15)
        def _stage_rem():
            pltpu.sync_copy(den0_hbm.at[pl.ds(16 * ROWS_PT, ROWS_REM)],
                            sh_den.at[pl.ds(16 * ROWS_PT, ROWS_REM)])
            pltpu.sync_copy(acc0_hbm.at[pl.ds(16 * ROWS_PT, ROWS_REM)],
                            sh_acc.at[pl.ds(16 * ROWS_PT, ROWS_REM)])

        plsc.subcore_barrier()

        iota = lax.iota(i32, 16)
        koff = iota >> 3
        kcol = iota & 7

        def gathers(k, p):
            return [
                pltpu.async_copy(asrc_hbm.at[src_buf.at[k]], a_s[p], gsem[p]),
                pltpu.async_copy(adst_hbm.at[dst_buf.at[k]], a_d[p], gsem[p]),
                pltpu.async_copy(h_hbm.at[src_buf.at[k]], hrows[p], gsem[p]),
            ]

        hcols = [16 * v + iota for v in range(F // 16)]
        wcols = [(16 * v + iota) >> lgG for v in range(F // 16)]

        def compute(p):
            @functools.partial(plsc.parallel_loop, 0, B * H // 16,
                               unroll=5, carry=koff)
            def _p1(j, rows):
                av = plsc.load_gather(a_s[p], [rows, kcol])
                dv = plsc.load_gather(a_d[p], [rows, kcol])
                plsc.store_scatter(w2d[p], [rows, kcol], _lrelu_exp(av + dv))
                return rows + 2

            @functools.partial(plsc.parallel_loop, 0, B,
                               unroll=2, carry=jnp.zeros((16,), i32))
            def _p2(e, esplat):
                for v in range(F // 16):
                    hv = plsc.load_gather(hrows[p], [esplat, hcols[v]])
                    wv = plsc.load_gather(w2d[p], [esplat, wcols[v]])
                    plsc.store_scatter(hrows[p], [esplat, hcols[v]], hv * wv)
                return esplat + 1

        def scatters(k, p):
            return [
                pltpu.async_copy(w2d[p], sh_den.at[dst_buf.at[k]], ssem[p],
                                 add=True),
                pltpu.async_copy(hrows[p], sh_acc.at[dst_buf.at[k]], ssem[p],
                                 add=True),
            ]

        def superblock(d, carry):
            pltpu.sync_copy(src_hbm.at[wid, d], src_buf)
            pltpu.sync_copy(dst_hbm.at[wid, d], dst_buf)
            g = {0: gathers(0, 0)}
            s = {}
            for k in range(NBS):
                p = k % 2
                for c in g.pop(k):
                    c.wait()
                if k >= 1:
                    for c in s.pop(k - 1):
                        c.wait()
                if k + 1 < NBS:
                    g[k + 1] = gathers(k + 1, (k + 1) % 2)
                compute(p)
                s[k] = scatters(k, p)
            for c in s.pop(NBS - 1):
                c.wait()
            return carry

        lax.fori_loop(0, NSB, superblock, 0)
        plsc.subcore_barrier()

        pltpu.sync_copy(sh_den.at[pl.ds(r0, ROWS_PT)],
                        den_out.at[cid, pl.ds(r0, ROWS_PT)])
        pltpu.sync_copy(sh_acc.at[pl.ds(r0, ROWS_PT)],
                        acc_out.at[cid, pl.ds(r0, ROWS_PT)])

        @pl.when(sid == 15)
        def _out_rem():
            pltpu.sync_copy(sh_den.at[pl.ds(16 * ROWS_PT, ROWS_REM)],
                            den_out.at[cid, pl.ds(16 * ROWS_PT, ROWS_REM)])
            pltpu.sync_copy(sh_acc.at[pl.ds(16 * ROWS_PT, ROWS_REM)],
                            acc_out.at[cid, pl.ds(16 * ROWS_PT, ROWS_REM)])

    return sc_layer


_sc1 = _make_sc_layer(F1)
_sc2 = _make_sc_layer(F2)


def _pack_a(a_flat, heads, ch):
    d = heads * ch
    rows = jnp.arange(d)[:, None] // ch
    mask = (rows == jnp.arange(8)[None, :]).astype(f32)
    return a_flat[:, None] * mask


def _expand_mat(heads, ch):
    d = heads * ch
    cols = jnp.arange(d)[None, :] // ch
    return (jnp.arange(8)[:, None] == cols).astype(f32)


def kernel(x, edge_index, W1, a_src1, a_dst1, W2, a_src2, a_dst2):
    src = edge_index[0].reshape(NTILES, NSB, NBS, B)
    dst = edge_index[1].reshape(NTILES, NSB, NBS, B)

    as1 = _pack_a(a_src1.reshape(-1), H, F1 // H)
    ad1 = _pack_a(a_dst1.reshape(-1), H, F1 // H)
    em1 = _expand_mat(H, F1 // H)
    as2 = jnp.tile(a_src2.reshape(F2, 1), (1, 8))
    ad2 = jnp.tile(a_dst2.reshape(F2, 1), (1, 8))
    em2 = _expand_mat(1, F2)[:8]

    h1, asrc1, adst1, den0h, acc0h = _tc1(x, W1, as1, ad1, em1)
    denp1, accp1 = _sc1(src, dst, asrc1, adst1, h1, den0h, acc0h)
    h2, asrc2, adst2, den02h, acc02h = _tc2(denp1, accp1, em1, W2, as2, ad2, em2)
    denp2, accp2 = _sc2(src, dst, asrc2, adst2, h2, den02h, acc02h)
    return _tc3(denp2, accp2, em2)[0]


# speedup vs baseline: 113.1608x; 1.0003x over previous
import functools

import jax
import jax.numpy as jnp
from jax import lax
from jax.experimental import pallas as pl
from jax.experimental.pallas import tpu as pltpu
from jax.experimental.pallas import tpu_sc as plsc

N = 10000
E = 320000
D_IN = 128
H = 8
F1 = 64
F2 = 128
NTILES = 32
EPT = E // NTILES
B = 100
NSB = 10
NBS = 10
NB = NSB * NBS
ROWS_PT = 624
ROWS_REM = N - 16 * ROWS_PT

f32 = jnp.float32
i32 = jnp.int32


def _lrelu_exp(s):
    return jnp.exp(jnp.where(s >= 0, s, 0.2 * s))


def _tc1_body(x_ref, w_ref, as_ref, ad_ref, em_ref,
              h_ref, asrc_ref, adst_ref, den0_ref, acc0_ref):
    h = jnp.dot(x_ref[...], w_ref[...], preferred_element_type=f32)
    asrc = jnp.dot(h, as_ref[...], preferred_element_type=f32)
    adst = jnp.dot(h, ad_ref[...], preferred_element_type=f32)
    ws = _lrelu_exp(asrc + adst)
    h_ref[...] = h
    asrc_ref[...] = asrc
    adst_ref[...] = adst
    den0_ref[...] = 0.5 * ws
    acc0_ref[...] = 0.5 * (jnp.dot(ws, em_ref[...], preferred_element_type=f32) * h)


def _tc2_body(denp_ref, accp_ref, em1_ref, w2_ref, as2_ref, ad2_ref, em2_ref,
              h2_ref, asrc_ref, adst_ref, den0_ref, acc0_ref):
    denp = denp_ref[...]
    accp = accp_ref[...]
    den = denp[0] + denp[1] + 1e-16
    acc = accp[0] + accp[1]
    out1 = acc / jnp.dot(den, em1_ref[...], preferred_element_type=f32)
    hh = jnp.where(out1 > 0, out1, jnp.exp(out1) - 1.0)
    h2 = jnp.dot(hh, w2_ref[...], preferred_element_type=f32)
    asrc = jnp.dot(h2, as2_ref[...], preferred_element_type=f32)
    adst = jnp.dot(h2, ad2_ref[...], preferred_element_type=f32)
    ws = _lrelu_exp(asrc + adst)
    h2_ref[...] = h2
    asrc_ref[...] = asrc
    adst_ref[...] = adst
    den0_ref[...] = 0.5 * ws
    acc0_ref[...] = 0.5 * (jnp.dot(ws, em2_ref[...], preferred_element_type=f32) * h2)


def _tc3_body(denp_ref, accp_ref, em2_ref, o_ref):
    denp = denp_ref[...]
    accp = accp_ref[...]
    den = denp[0] + denp[1] + 1e-16
    acc = accp[0] + accp[1]
    out = acc / jnp.dot(den, em2_ref[...], preferred_element_type=f32)
    m = jnp.max(out, axis=-1, keepdims=True)
    z = out - m
    lse = jnp.log(jnp.sum(jnp.exp(z), axis=-1, keepdims=True))
    o_ref[...] = z - lse


_R = 2000


def _rows(shape):
    return pl.BlockSpec((_R, shape), lambda i: (i, 0))


def _full(*shape):
    return pl.BlockSpec(shape, lambda i: (0,) * len(shape))


def _prow(k):
    return pl.BlockSpec((2, _R, k), lambda i: (0, i, 0))


def _tc1(x, w1, as1, ad1, em1):
    return pl.pallas_call(
        _tc1_body,
        grid=(N // _R,),
        in_specs=[_rows(D_IN), _full(D_IN, F1), _full(F1, H), _full(F1, H),
                  _full(H, F1)],
        out_specs=[_rows(F1), _rows(H), _rows(H), _rows(H), _rows(F1)],
        out_shape=[
            jax.ShapeDtypeStruct((N, F1), f32),
            jax.ShapeDtypeStruct((N, H), f32),
            jax.ShapeDtypeStruct((N, H), f32),
            jax.ShapeDtypeStruct((N, H), f32),
            jax.ShapeDtypeStruct((N, F1), f32),
        ],
    )(x, w1, as1, ad1, em1)


def _tc2(denp, accp, em1, w2, as2, ad2, em2):
    return pl.pallas_call(
        _tc2_body,
        grid=(N // _R,),
        in_specs=[_prow(H), _prow(F1), _full(H, F1), _full(F1, F2),
                  _full(F2, H), _full(F2, H), _full(H, F2)],
        out_specs=[_rows(F2), _rows(H), _rows(H), _rows(H), _rows(F2)],
        out_shape=[
            jax.ShapeDtypeStruct((N, F2), f32),
            jax.ShapeDtypeStruct((N, H), f32),
            jax.ShapeDtypeStruct((N, H), f32),
            jax.ShapeDtypeStruct((N, H), f32),
            jax.ShapeDtypeStruct((N, F2), f32),
        ],
    )(denp, accp, em1, w2, as2, ad2, em2)


def _tc3(denp, accp, em2):
    return pl.pallas_call(
        _tc3_body,
        grid=(N // _R,),
        in_specs=[_prow(H), _prow(F2), _full(H, F2)],
        out_specs=[_rows(F2)],
        out_shape=[jax.ShapeDtypeStruct((N, F2), f32)],
    )(denp, accp, em2)


def _make_sc_layer(F):
    lgF = F.bit_length() - 1
    lgG = (F // 8).bit_length() - 1

    mesh = plsc.VectorSubcoreMesh(core_axis_name="c", subcore_axis_name="s")

    @functools.partial(
        pl.kernel,
        mesh=mesh,
        compiler_params=pltpu.CompilerParams(
            needs_layout_passes=False,
            use_tc_tiling_on_sc=False,
        ),
        out_type=[
            jax.ShapeDtypeStruct((2, N, H), f32),
            jax.ShapeDtypeStruct((2, N, F), f32),
        ],
        scratch_types=[
            pltpu.VMEM_SHARED((N, H), f32),
            pltpu.VMEM_SHARED((N, F), f32),
            pltpu.VMEM((NBS, B), i32),
            pltpu.VMEM((NBS, B), i32),
            [pltpu.VMEM((B, H), f32)] * 2,
            [pltpu.VMEM((B, H), f32)] * 2,
            [pltpu.VMEM((B, H), f32)] * 2,
            [pltpu.VMEM((B, F), f32)] * 2,
            [pltpu.SemaphoreType.DMA] * 2,
            [pltpu.SemaphoreType.DMA] * 2,
        ],
    )
    def sc_layer(src_hbm, dst_hbm, asrc_hbm, adst_hbm, h_hbm, den0_hbm,
                 acc0_hbm, den_out, acc_out,
                 sh_den, sh_acc, src_buf, dst_buf, a_s, a_d, w2d, hrows,
                 gsem, ssem):
        cid = lax.axis_index("c")
        sid = lax.axis_index("s")
        wid = cid * 16 + sid
        r0 = sid * ROWS_PT

        pltpu.sync_copy(den0_hbm.at[pl.ds(r0, ROWS_PT)],
                        sh_den.at[pl.ds(r0, ROWS_PT)])
        pltpu.sync_copy(acc0_hbm.at[pl.ds(r0, ROWS_PT)],
                        sh_acc.at[pl.ds(r0, ROWS_PT)])

        @pl.when(sid == 15)
        def _stage_rem():
            pltpu.sync_copy(den0_hbm.at[pl.ds(16 * ROWS_PT, ROWS_REM)],
                            sh_den.at[pl.ds(16 * ROWS_PT, ROWS_REM)])
            pltpu.sync_copy(acc0_hbm.at[pl.ds(16 * ROWS_PT, ROWS_REM)],
                            sh_acc.at[pl.ds(16 * ROWS_PT, ROWS_REM)])

        plsc.subcore_barrier()

        iota = lax.iota(i32, 16)
        koff = iota >> 3
        kcol = iota & 7

        def gathers(k, p):
            return [
                pltpu.async_copy(asrc_hbm.at[src_buf.at[k]], a_s[p], gsem[p]),
                pltpu.async_copy(adst_hbm.at[dst_buf.at[k]], a_d[p], gsem[p]),
                pltpu.async_copy(h_hbm.at[src_buf.at[k]], hrows[p], gsem[p]),
            ]

        hcols = [16 * v + iota for v in range(F // 16)]
        wcols = [(16 * v + iota) >> lgG for v in range(F // 16)]

        def compute(p):
            @functools.partial(plsc.parallel_loop, 0, B * H // 16,
                               unroll=5, carry=koff)
            def _p1(j, rows):
                av = plsc.load_gather(a_s[p], [rows, kcol])
                dv = plsc.load_gather(a_d[p], [rows, kcol])
                plsc.store_scatter(w2d[p], [rows, kcol], _lrelu_exp(av + dv))
                return rows + 2

            @functools.partial(plsc.parallel_loop, 0, B,
                               unroll=4, carry=jnp.zeros((16,), i32))
            def _p2(e, esplat):
                if F == 128:
                    wv = plsc.load_gather(w2d[p], [esplat, jnp.zeros((16,), i32)])
                for v in range(F // 16):
                    hv = plsc.load_gather(hrows[p], [esplat, hcols[v]])
                    if F != 128:
                        wv = plsc.load_gather(w2d[p], [esplat, wcols[v]])
                    plsc.store_scatter(hrows[p], [esplat, hcols[v]], hv * wv)
                return esplat + 1

        def scatters(k, p):
            return [
                pltpu.async_copy(w2d[p], sh_den.at[dst_buf.at[k]], ssem[p],
                                 add=True),
                pltpu.async_copy(hrows[p], sh_acc.at[dst_buf.at[k]], ssem[p],
                                 add=True),
            ]

        def superblock(d, carry):
            pltpu.sync_copy(src_hbm.at[wid, d], src_buf)
            pltpu.sync_copy(dst_hbm.at[wid, d], dst_buf)
            g = {0: gathers(0, 0)}
            s = {}
            for k in range(NBS):
                p = k % 2
                for c in g.pop(k):
                    c.wait()
                if k >= 1:
                    for c in s.pop(k - 1):
                        c.wait()
                if k + 1 < NBS:
                    g[k + 1] = gathers(k + 1, (k + 1) % 2)
                compute(p)
                s[k] = scatters(k, p)
            for c in s.pop(NBS - 1):
                c.wait()
            return carry

        lax.fori_loop(0, NSB, superblock, 0)
        plsc.subcore_barrier()

        pltpu.sync_copy(sh_den.at[pl.ds(r0, ROWS_PT)],
                        den_out.at[cid, pl.ds(r0, ROWS_PT)])
        pltpu.sync_copy(sh_acc.at[pl.ds(r0, ROWS_PT)],
                        acc_out.at[cid, pl.ds(r0, ROWS_PT)])

        @pl.when(sid == 15)
        def _out_rem():
            pltpu.sync_copy(sh_den.at[pl.ds(16 * ROWS_PT, ROWS_REM)],
                            den_out.at[cid, pl.ds(16 * ROWS_PT, ROWS_REM)])
            pltpu.sync_copy(sh_acc.at[pl.ds(16 * ROWS_PT, ROWS_REM)],
                            acc_out.at[cid, pl.ds(16 * ROWS_PT, ROWS_REM)])

    return sc_layer


_sc1 = _make_sc_layer(F1)
_sc2 = _make_sc_layer(F2)


def _pack_a(a_flat, heads, ch):
    d = heads * ch
    rows = jnp.arange(d)[:, None] // ch
    mask = (rows == jnp.arange(8)[None, :]).astype(f32)
    return a_flat[:, None] * mask


def _expand_mat(heads, ch):
    d = heads * ch
    cols = jnp.arange(d)[None, :] // ch
    return (jnp.arange(8)[:, None] == cols).astype(f32)


def kernel(x, edge_index, W1, a_src1, a_dst1, W2, a_src2, a_dst2):
    src = edge_index[0].reshape(NTILES, NSB, NBS, B)
    dst = edge_index[1].reshape(NTILES, NSB, NBS, B)

    as1 = _pack_a(a_src1.reshape(-1), H, F1 // H)
    ad1 = _pack_a(a_dst1.reshape(-1), H, F1 // H)
    em1 = _expand_mat(H, F1 // H)
    as2 = jnp.tile(a_src2.reshape(F2, 1), (1, 8))
    ad2 = jnp.tile(a_dst2.reshape(F2, 1), (1, 8))
    em2 = _expand_mat(1, F2)[:8]

    h1, asrc1, adst1, den0h, acc0h = _tc1(x, W1, as1, ad1, em1)
    denp1, accp1 = _sc1(src, dst, asrc1, adst1, h1, den0h, acc0h)
    h2, asrc2, adst2, den02h, acc02h = _tc2(denp1, accp1, em1, W2, as2, ad2, em2)
    denp2, accp2 = _sc2(src, dst, asrc2, adst2, h2, den02h, acc02h)
    return _tc3(denp2, accp2, em2)[0]
